# Initial kernel scaffold; baseline (speedup 1.0000x reference)
import jax
import jax.numpy as jnp
from jax.experimental import pallas as pl


def kernel(x, edge_index, W1, b1, W2, b2, fcW, fcb):
    raise NotImplementedError("write your pallas kernel here")


# speedup vs baseline: 12.6366x; 12.6366x over previous
import functools

import jax
import jax.numpy as jnp
from jax import lax
from jax.experimental import pallas as pl
from jax.experimental.pallas import tpu as pltpu
from jax.experimental.pallas import tpu_sc as plsc

_N = 10000
_F = 128
_NPAD = 10240
_NW = 32
_CHUNK = 128


def _sc_mesh():
    return plsc.VectorSubcoreMesh(core_axis_name="c", subcore_axis_name="s")


_DEGW = 128


def _sc_degree(dst3, ones_col, zeros_col):
    cpt = dst3.shape[1]
    zrows = _NPAD // 16

    @functools.partial(
        pl.kernel,
        out_type=jax.ShapeDtypeStruct((2, _NPAD, _DEGW), jnp.float32),
        mesh=_sc_mesh(),
        scratch_types=[
            pltpu.VMEM((cpt, _CHUNK), jnp.int32),
            pltpu.VMEM((_CHUNK, _DEGW), jnp.float32),
            pltpu.VMEM_SHARED((_NPAD, _DEGW), jnp.float32),
        ],
    )
    def k(dst_h, ones_h, zeros_h, out_h, dst_v, ones_v, acc_sh):
        c = lax.axis_index("c")
        s = lax.axis_index("s")
        wid = s * 2 + c
        zone = pl.ds(s * zrows, zrows)
        pltpu.sync_copy(zeros_h, acc_sh.at[zone])
        pltpu.sync_copy(dst_h.at[wid], dst_v)
        pltpu.sync_copy(ones_h, ones_v)
        plsc.subcore_barrier()

        def body(j, _):
            pltpu.sync_copy(ones_v, acc_sh.at[dst_v.at[j]], add=True)
            return ()

        lax.fori_loop(0, cpt, body, ())
        plsc.subcore_barrier()
        pltpu.sync_copy(acc_sh.at[zone], out_h.at[c, zone])

    return k(dst3, ones_col, zeros_col)


def _sc_aggregate(table, src3, dst3, zeros_row):
    cpt = src3.shape[1]
    zrows = _NPAD // 16

    @functools.partial(
        pl.kernel,
        out_type=jax.ShapeDtypeStruct((2, _NPAD, _F), jnp.float32),
        mesh=_sc_mesh(),
        scratch_types=[
            pltpu.VMEM((cpt, _CHUNK), jnp.int32),
            pltpu.VMEM((cpt, _CHUNK), jnp.int32),
            pltpu.VMEM((_CHUNK, _F), jnp.float32),
            pltpu.VMEM_SHARED((_NPAD, _F), jnp.float32),
            pltpu.SemaphoreType.DMA,
        ],
    )
    def k(tab_h, src_h, dst_h, zeros_h, out_h, src_v, dst_v, msg_v, acc_sh, sem):
        c = lax.axis_index("c")
        s = lax.axis_index("s")
        wid = s * 2 + c
        zone = pl.ds(s * zrows, zrows)
        pltpu.sync_copy(zeros_h, acc_sh.at[zone])
        pltpu.sync_copy(src_h.at[wid], src_v)
        pltpu.sync_copy(dst_h.at[wid], dst_v)
        plsc.subcore_barrier()

        def body(j, _):
            pltpu.async_copy(tab_h.at[src_v.at[j]], msg_v, sem).wait()
            pltpu.sync_copy(msg_v, acc_sh.at[dst_v.at[j]], add=True)
            return ()

        lax.fori_loop(0, cpt, body, ())
        plsc.subcore_barrier()
        pltpu.sync_copy(acc_sh.at[zone], out_h.at[c, zone])

    return k(table, src3, dst3, zeros_row)


def _tc_first(x, W1, deg2):
    def body(x_ref, w_ref, deg_ref, dinv_ref, a_ref, hs_ref):
        deg = deg_ref[0, :, 0:1] + deg_ref[1, :, 0:1] + 1.0
        dinv = lax.rsqrt(deg)
        dinv_ref[...] = dinv
        a = lax.dot_general(x_ref[...], w_ref[...], (((1,), (1,)), ((), ())),
                            preferred_element_type=jnp.float32,
                            precision=lax.Precision.HIGHEST)
        a_ref[...] = a
        hs_ref[...] = a * dinv[:_N]

    return pl.pallas_call(
        body,
        out_shape=[
            jax.ShapeDtypeStruct((_NPAD, 1), jnp.float32),
            jax.ShapeDtypeStruct((_N, _F), jnp.float32),
            jax.ShapeDtypeStruct((_N, _F), jnp.float32),
        ],
    )(x, W1, deg2)


def _tc_mid(S, a_prev, dinv, b_prev, W):
    def body(s_ref, a_ref, dinv_ref, b_ref, w_ref, an_ref, hs_ref):
        dv = dinv_ref[:_N]
        agg = s_ref[0, : _N, :] + s_ref[1, : _N, :]
        h = dv * agg + (dv * dv) * a_ref[...] + b_ref[...]
        h = jnp.maximum(h, 0.0)
        a_next = lax.dot_general(h, w_ref[...], (((1,), (1,)), ((), ())),
                                 preferred_element_type=jnp.float32,
                                 precision=lax.Precision.HIGHEST)
        an_ref[...] = a_next
        hs_ref[...] = a_next * dv

    return pl.pallas_call(
        body,
        out_shape=[
            jax.ShapeDtypeStruct((_N, _F), jnp.float32),
            jax.ShapeDtypeStruct((_N, _F), jnp.float32),
        ],
    )(S, a_prev, dinv, b_prev, W)


def _tc_last(S, a_prev, dinv, b_prev, fcW, fcb):
    def body(s_ref, a_ref, dinv_ref, b_ref, w_ref, fb_ref, o_ref):
        dv = dinv_ref[:_N]
        agg = s_ref[0, : _N, :] + s_ref[1, : _N, :]
        h = dv * agg + (dv * dv) * a_ref[...] + b_ref[...]
        h = jnp.maximum(h, 0.0)
        o_ref[...] = lax.dot_general(h, w_ref[...], (((1,), (1,)), ((), ())),
                                     preferred_element_type=jnp.float32,
                                     precision=lax.Precision.HIGHEST) + fb_ref[...]

    return pl.pallas_call(
        body,
        out_shape=jax.ShapeDtypeStruct((_N, fcW.shape[0]), jnp.float32),
    )(S, a_prev, dinv, b_prev, fcW, fcb)


def kernel(x, edge_index, W1, b1, W2, b2, fcW, fcb):
    E = edge_index.shape[1]
    epad = -(-E // (_NW * _CHUNK)) * (_NW * _CHUNK)
    cpt = epad // (_NW * _CHUNK)

    src = edge_index[0]
    dst = edge_index[1]
    src3 = jnp.concatenate(
        [src, jnp.zeros((epad - E,), jnp.int32)]).reshape(_NW, cpt, _CHUNK)
    dst3 = jnp.concatenate(
        [dst, jnp.full((epad - E,), _N, jnp.int32)]).reshape(_NW, cpt, _CHUNK)

    zrows = _NPAD // 16
    ones_col = jnp.ones((_CHUNK, _DEGW), jnp.float32)
    zeros_col = jnp.zeros((zrows, _DEGW), jnp.float32)
    zeros_row = jnp.zeros((zrows, _F), jnp.float32)

    deg2 = _sc_degree(dst3, ones_col, zeros_col)
    dinv, a1, hs1 = _tc_first(x, W1, deg2)
    S1 = _sc_aggregate(hs1, src3, dst3, zeros_row)
    a2, hs2 = _tc_mid(S1, a1, dinv, b1, W2)
    S2 = _sc_aggregate(hs2, src3, dst3, zeros_row)
    return _tc_last(S2, a2, dinv, b2, fcW, fcb)
